# Initial kernel scaffold; baseline (speedup 1.0000x reference)
import jax
import jax.numpy as jnp
from jax.experimental import pallas as pl


def kernel(x, ln1_g, ln1_b, W_attn, b_attn, W_proj, b_proj, ln2_g, ln2_b, W_r, W1, b1, W2, b2):
    raise NotImplementedError("write your pallas kernel here")


# speedup vs baseline: 1.0033x; 1.0033x over previous
import functools

import jax
import jax.numpy as jnp
from jax.experimental import pallas as pl
from jax.experimental.pallas import tpu as pltpu

B, T, C = 1, 2048, 2048
NH = 16
DH = C // NH
E = 8
HID = 4096
TOPK = 2

BM = 256
BN_QKV = 768
BN_PROJ = 512
BH = 512

NEG = -1e30


def _gelu_exact(a):
    return 0.5 * a * (1.0 + jax.lax.erf(a * (2.0 ** -0.5)))


def _ln(xb, g, b):
    mu = jnp.mean(xb, axis=-1, keepdims=True)
    var = jnp.mean((xb - mu) ** 2, axis=-1, keepdims=True)
    return (xb - mu) / jnp.sqrt(var + 1e-5) * g + b


def _ln_mm_kernel(x_ref, g_ref, b_ref, w_ref, bias_ref, o_ref):
    h = _ln(x_ref[...], g_ref[...], b_ref[...])
    o_ref[...] = (
        jnp.dot(h, w_ref[...], preferred_element_type=jnp.float32)
        + bias_ref[...]
    )


def _ln_matmul(x, g, b, w, bias, bn):
    n = w.shape[1]
    grid = (T // BM, n // bn)
    return pl.pallas_call(
        _ln_mm_kernel,
        grid=grid,
        in_specs=[
            pl.BlockSpec((BM, C), lambda i, j: (i, 0)),
            pl.BlockSpec((1, C), lambda i, j: (0, 0)),
            pl.BlockSpec((1, C), lambda i, j: (0, 0)),
            pl.BlockSpec((C, bn), lambda i, j: (0, j)),
            pl.BlockSpec((1, bn), lambda i, j: (0, j)),
        ],
        out_specs=pl.BlockSpec((BM, bn), lambda i, j: (i, j)),
        out_shape=jax.ShapeDtypeStruct((T, n), jnp.float32),
    )(x, g.reshape(1, C), b.reshape(1, C), w, bias.reshape(1, n))


def _attn_kernel(q_ref, k_ref, v_ref, o_ref):
    i = pl.program_id(1)
    q = q_ref[0]
    k = k_ref[0]
    v = v_ref[0]
    s = jax.lax.dot_general(
        q, k, (((1,), (1,)), ((), ())), preferred_element_type=jnp.float32
    ) * (1.0 / (DH ** 0.5))
    rows = i * BM + jax.lax.broadcasted_iota(jnp.int32, (BM, T), 0)
    cols = jax.lax.broadcasted_iota(jnp.int32, (BM, T), 1)
    s = jnp.where(cols <= rows, s, NEG)
    m = jnp.max(s, axis=-1, keepdims=True)
    pu = jnp.exp(s - m)
    den = jnp.sum(pu, axis=-1, keepdims=True)
    o_ref[0] = jnp.dot(pu, v, preferred_element_type=jnp.float32) / den


def _attention(q, k, v):
    grid = (NH, T // BM)
    return pl.pallas_call(
        _attn_kernel,
        grid=grid,
        in_specs=[
            pl.BlockSpec((1, BM, DH), lambda h, i: (h, i, 0)),
            pl.BlockSpec((1, T, DH), lambda h, i: (h, 0, 0)),
            pl.BlockSpec((1, T, DH), lambda h, i: (h, 0, 0)),
        ],
        out_specs=pl.BlockSpec((1, BM, DH), lambda h, i: (h, i, 0)),
        out_shape=jax.ShapeDtypeStruct((NH, T, DH), jnp.float32),
    )(q, k, v)


def _mm_res_kernel(y_ref, w_ref, b_ref, x_ref, o_ref):
    o_ref[...] = (
        jnp.dot(y_ref[...], w_ref[...], preferred_element_type=jnp.float32)
        + b_ref[...]
        + x_ref[...]
    )


def _proj_residual(y, w, b, x):
    grid = (T // BM, C // BN_PROJ)
    return pl.pallas_call(
        _mm_res_kernel,
        grid=grid,
        in_specs=[
            pl.BlockSpec((BM, C), lambda i, j: (i, 0)),
            pl.BlockSpec((C, BN_PROJ), lambda i, j: (0, j)),
            pl.BlockSpec((1, BN_PROJ), lambda i, j: (0, j)),
            pl.BlockSpec((BM, BN_PROJ), lambda i, j: (i, j)),
        ],
        out_specs=pl.BlockSpec((BM, BN_PROJ), lambda i, j: (i, j)),
        out_shape=jax.ShapeDtypeStruct((T, C), jnp.float32),
    )(y, w, b.reshape(1, C), x)


def _ln_kernel(x_ref, g_ref, b_ref, o_ref):
    o_ref[...] = _ln(x_ref[...], g_ref[...], b_ref[...])


def _layernorm2(x, g, b):
    return pl.pallas_call(
        _ln_kernel,
        grid=(T // BM,),
        in_specs=[
            pl.BlockSpec((BM, C), lambda i: (i, 0)),
            pl.BlockSpec((1, C), lambda i: (0, 0)),
            pl.BlockSpec((1, C), lambda i: (0, 0)),
        ],
        out_specs=pl.BlockSpec((BM, C), lambda i: (i, 0)),
        out_shape=jax.ShapeDtypeStruct((T, C), jnp.float32),
    )(x, g.reshape(1, C), b.reshape(1, C))


def _router_kernel(h_ref, wr_ref, w8_ref, i1_ref, i2_ref, w1_ref, w2_ref,
                   cnt_ref):
    logits = jnp.dot(h_ref[...], wr_ref[...],
                     preferred_element_type=jnp.float32)
    lane = jax.lax.broadcasted_iota(jnp.int32, (T, 128), 1)
    valid = lane < E
    logits = jnp.where(valid, logits, NEG)
    mx = jnp.max(logits, axis=-1, keepdims=True)
    ex = jnp.exp(logits - mx)
    prob = ex / jnp.sum(ex, axis=-1, keepdims=True)

    big = jnp.int32(10**9)
    m1 = jnp.max(prob, axis=-1, keepdims=True)
    i1 = jnp.min(jnp.where(valid & (prob == m1), lane, big), axis=-1,
                 keepdims=True)
    p2 = jnp.where(lane == i1, -1.0, prob)
    m2 = jnp.max(p2, axis=-1, keepdims=True)
    i2 = jnp.min(jnp.where(valid & (p2 == m2), lane, big), axis=-1,
                 keepdims=True)

    denom = m1 + m2 + 1e-9
    w1 = m1 / denom
    w2 = m2 / denom
    sel1 = lane == i1
    sel2 = lane == i2
    w8_ref[...] = jnp.where(sel1, w1, 0.0) + jnp.where(sel2, w2, 0.0)
    i1_ref[...] = jnp.broadcast_to(i1, (T, 128))
    i2_ref[...] = jnp.broadcast_to(i2, (T, 128))
    w1_ref[...] = jnp.broadcast_to(w1, (T, 128))
    w2_ref[...] = jnp.broadcast_to(w2, (T, 128))
    onehot = sel1.astype(jnp.float32) + sel2.astype(jnp.float32)
    cnt_ref[...] = jnp.sum(onehot, axis=0, keepdims=True)


def _router(h2, w_r):
    wr_pad = jnp.zeros((C, 128), jnp.float32).at[:, :E].set(w_r)
    outs = pl.pallas_call(
        _router_kernel,
        grid=(1,),
        in_specs=[
            pl.BlockSpec((T, C), lambda i: (0, 0)),
            pl.BlockSpec((C, 128), lambda i: (0, 0)),
        ],
        out_specs=[
            pl.BlockSpec((T, 128), lambda i: (0, 0)),
            pl.BlockSpec((T, 128), lambda i: (0, 0)),
            pl.BlockSpec((T, 128), lambda i: (0, 0)),
            pl.BlockSpec((T, 128), lambda i: (0, 0)),
            pl.BlockSpec((T, 128), lambda i: (0, 0)),
            pl.BlockSpec((1, 128), lambda i: (0, 0)),
        ],
        out_shape=[
            jax.ShapeDtypeStruct((T, 128), jnp.float32),
            jax.ShapeDtypeStruct((T, 128), jnp.int32),
            jax.ShapeDtypeStruct((T, 128), jnp.int32),
            jax.ShapeDtypeStruct((T, 128), jnp.float32),
            jax.ShapeDtypeStruct((T, 128), jnp.float32),
            jax.ShapeDtypeStruct((1, 128), jnp.float32),
        ],
    )(h2, wr_pad)
    return outs


def _moe_dense_kernel(h_ref, w1_ref, b1_ref, w2_ref, b2_ref, wbc_ref, x_ref,
                      o_ref):
    e = pl.program_id(1)
    h = pl.program_id(2)

    @pl.when((e == 0) & (h == 0))
    def _init():
        o_ref[...] = x_ref[...]

    wcol = wbc_ref[0][:, 0:1]

    @pl.when(h == 0)
    def _bias2():
        o_ref[...] += wcol * b2_ref[0]

    a = jnp.dot(h_ref[...], w1_ref[0],
                preferred_element_type=jnp.float32) + b1_ref[0]
    a = _gelu_exact(a) * wcol
    o_ref[...] += jnp.dot(a, w2_ref[0], preferred_element_type=jnp.float32)


def _moe_dense(h2, W1, b1, W2, b2, w8, x2):
    wbc = jnp.broadcast_to(w8[:, :E].T[:, :, None], (E, T, 128))
    wbc = jnp.asarray(wbc)
    grid = (T // BM, E, HID // BH)
    return pl.pallas_call(
        _moe_dense_kernel,
        grid=grid,
        in_specs=[
            pl.BlockSpec((BM, C), lambda m, e, h: (m, 0)),
            pl.BlockSpec((1, C, BH), lambda m, e, h: (e, 0, h)),
            pl.BlockSpec((1, 1, BH), lambda m, e, h: (e, 0, h)),
            pl.BlockSpec((1, BH, C), lambda m, e, h: (e, h, 0)),
            pl.BlockSpec((1, 1, C), lambda m, e, h: (e, 0, 0)),
            pl.BlockSpec((1, BM, 128), lambda m, e, h: (e, m, 0)),
            pl.BlockSpec((BM, C), lambda m, e, h: (m, 0)),
        ],
        out_specs=pl.BlockSpec((BM, C), lambda m, e, h: (m, 0)),
        out_shape=jax.ShapeDtypeStruct((T, C), jnp.float32),
    )(h2, W1, b1.reshape(E, 1, HID), W2, b2.reshape(E, 1, C), wbc, x2)


def kernel(x, ln1_g, ln1_b, W_attn, b_attn, W_proj, b_proj, ln2_g, ln2_b,
           W_r, W1, b1, W2, b2):
    xf = x.reshape(T, C)
    qkv = _ln_matmul(xf, ln1_g, ln1_b, W_attn, b_attn, BN_QKV)
    q, k, v = jnp.split(qkv, 3, axis=1)
    q = q.reshape(T, NH, DH).transpose(1, 0, 2)
    k = k.reshape(T, NH, DH).transpose(1, 0, 2)
    v = v.reshape(T, NH, DH).transpose(1, 0, 2)
    y = _attention(q, k, v)
    y = y.transpose(1, 0, 2).reshape(T, C)
    x2 = _proj_residual(y, W_proj, b_proj, xf)

    h2 = _layernorm2(x2, ln2_g, ln2_b)
    w8, i1b, i2b, w1b, w2b, cnt = _router(h2, W_r)
    out = _moe_dense(h2, W1, b1, W2, b2, w8, x2)

    counts = cnt[0, :E]
    util = counts / (jnp.sum(counts) + 1e-9)
    return out.reshape(B, T, C), util


# speedup vs baseline: 1.5490x; 1.5438x over previous
import functools

import jax
import jax.numpy as jnp
from jax.experimental import pallas as pl
from jax.experimental.pallas import tpu as pltpu

B, T, C = 1, 2048, 2048
NH = 16
DH = C // NH
E = 8
HID = 4096
TOPK = 2

BM = 256
BN_QKV = 768
BN_PROJ = 512
BH = 512

NEG = -1e30


def _gelu_exact(a):
    return 0.5 * a * (1.0 + jax.lax.erf(a * (2.0 ** -0.5)))


def _ln(xb, g, b):
    mu = jnp.mean(xb, axis=-1, keepdims=True)
    var = jnp.mean((xb - mu) ** 2, axis=-1, keepdims=True)
    return (xb - mu) / jnp.sqrt(var + 1e-5) * g + b


def _ln_mm_kernel(x_ref, g_ref, b_ref, w_ref, bias_ref, o_ref):
    h = _ln(x_ref[...], g_ref[...], b_ref[...])
    o_ref[...] = (
        jnp.dot(h, w_ref[...], preferred_element_type=jnp.float32)
        + bias_ref[...]
    )


def _ln_matmul(x, g, b, w, bias, bn):
    n = w.shape[1]
    grid = (T // BM, n // bn)
    return pl.pallas_call(
        _ln_mm_kernel,
        grid=grid,
        in_specs=[
            pl.BlockSpec((BM, C), lambda i, j: (i, 0)),
            pl.BlockSpec((1, C), lambda i, j: (0, 0)),
            pl.BlockSpec((1, C), lambda i, j: (0, 0)),
            pl.BlockSpec((C, bn), lambda i, j: (0, j)),
            pl.BlockSpec((1, bn), lambda i, j: (0, j)),
        ],
        out_specs=pl.BlockSpec((BM, bn), lambda i, j: (i, j)),
        out_shape=jax.ShapeDtypeStruct((T, n), jnp.float32),
    )(x, g.reshape(1, C), b.reshape(1, C), w, bias.reshape(1, n))


def _attn_kernel(q_ref, k_ref, v_ref, o_ref):
    i = pl.program_id(1)
    q = q_ref[0]
    k = k_ref[0]
    v = v_ref[0]
    s = jax.lax.dot_general(
        q, k, (((1,), (1,)), ((), ())), preferred_element_type=jnp.float32
    ) * (1.0 / (DH ** 0.5))
    rows = i * BM + jax.lax.broadcasted_iota(jnp.int32, (BM, T), 0)
    cols = jax.lax.broadcasted_iota(jnp.int32, (BM, T), 1)
    s = jnp.where(cols <= rows, s, NEG)
    m = jnp.max(s, axis=-1, keepdims=True)
    pu = jnp.exp(s - m)
    den = jnp.sum(pu, axis=-1, keepdims=True)
    o_ref[0] = jnp.dot(pu, v, preferred_element_type=jnp.float32) / den


def _attention(q, k, v):
    grid = (NH, T // BM)
    return pl.pallas_call(
        _attn_kernel,
        grid=grid,
        in_specs=[
            pl.BlockSpec((1, BM, DH), lambda h, i: (h, i, 0)),
            pl.BlockSpec((1, T, DH), lambda h, i: (h, 0, 0)),
            pl.BlockSpec((1, T, DH), lambda h, i: (h, 0, 0)),
        ],
        out_specs=pl.BlockSpec((1, BM, DH), lambda h, i: (h, i, 0)),
        out_shape=jax.ShapeDtypeStruct((NH, T, DH), jnp.float32),
    )(q, k, v)


def _mm_res_kernel(y_ref, w_ref, b_ref, x_ref, o_ref):
    o_ref[...] = (
        jnp.dot(y_ref[...], w_ref[...], preferred_element_type=jnp.float32)
        + b_ref[...]
        + x_ref[...]
    )


def _proj_residual(y, w, b, x):
    grid = (T // BM, C // BN_PROJ)
    return pl.pallas_call(
        _mm_res_kernel,
        grid=grid,
        in_specs=[
            pl.BlockSpec((BM, C), lambda i, j: (i, 0)),
            pl.BlockSpec((C, BN_PROJ), lambda i, j: (0, j)),
            pl.BlockSpec((1, BN_PROJ), lambda i, j: (0, j)),
            pl.BlockSpec((BM, BN_PROJ), lambda i, j: (i, j)),
        ],
        out_specs=pl.BlockSpec((BM, BN_PROJ), lambda i, j: (i, j)),
        out_shape=jax.ShapeDtypeStruct((T, C), jnp.float32),
    )(y, w, b.reshape(1, C), x)


def _ln_kernel(x_ref, g_ref, b_ref, o_ref):
    o_ref[...] = _ln(x_ref[...], g_ref[...], b_ref[...])


def _layernorm2(x, g, b):
    return pl.pallas_call(
        _ln_kernel,
        grid=(T // BM,),
        in_specs=[
            pl.BlockSpec((BM, C), lambda i: (i, 0)),
            pl.BlockSpec((1, C), lambda i: (0, 0)),
            pl.BlockSpec((1, C), lambda i: (0, 0)),
        ],
        out_specs=pl.BlockSpec((BM, C), lambda i: (i, 0)),
        out_shape=jax.ShapeDtypeStruct((T, C), jnp.float32),
    )(x, g.reshape(1, C), b.reshape(1, C))


def _router_kernel(h_ref, wr_ref, w8_ref, i1_ref, i2_ref, w1_ref, w2_ref,
                   cnt_ref):
    logits = jnp.dot(h_ref[...], wr_ref[...],
                     preferred_element_type=jnp.float32)
    lane = jax.lax.broadcasted_iota(jnp.int32, (T, 128), 1)
    valid = lane < E
    logits = jnp.where(valid, logits, NEG)
    mx = jnp.max(logits, axis=-1, keepdims=True)
    ex = jnp.exp(logits - mx)
    prob = ex / jnp.sum(ex, axis=-1, keepdims=True)

    big = jnp.int32(10**9)
    m1 = jnp.max(prob, axis=-1, keepdims=True)
    i1 = jnp.min(jnp.where(valid & (prob == m1), lane, big), axis=-1,
                 keepdims=True)
    p2 = jnp.where(lane == i1, -1.0, prob)
    m2 = jnp.max(p2, axis=-1, keepdims=True)
    i2 = jnp.min(jnp.where(valid & (p2 == m2), lane, big), axis=-1,
                 keepdims=True)

    denom = m1 + m2 + 1e-9
    w1 = m1 / denom
    w2 = m2 / denom
    sel1 = lane == i1
    sel2 = lane == i2
    w8_ref[...] = jnp.where(sel1, w1, 0.0) + jnp.where(sel2, w2, 0.0)
    i1_ref[...] = jnp.broadcast_to(i1, (T, 128))
    i2_ref[...] = jnp.broadcast_to(i2, (T, 128))
    w1_ref[...] = jnp.broadcast_to(w1, (T, 128))
    w2_ref[...] = jnp.broadcast_to(w2, (T, 128))
    onehot = sel1.astype(jnp.float32) + sel2.astype(jnp.float32)
    cnt_ref[...] = jnp.sum(onehot, axis=0, keepdims=True)


def _router(h2, w_r):
    wr_pad = jnp.zeros((C, 128), jnp.float32).at[:, :E].set(w_r)
    outs = pl.pallas_call(
        _router_kernel,
        grid=(1,),
        in_specs=[
            pl.BlockSpec((T, C), lambda i: (0, 0)),
            pl.BlockSpec((C, 128), lambda i: (0, 0)),
        ],
        out_specs=[
            pl.BlockSpec((T, 128), lambda i: (0, 0)),
            pl.BlockSpec((T, 128), lambda i: (0, 0)),
            pl.BlockSpec((T, 128), lambda i: (0, 0)),
            pl.BlockSpec((T, 128), lambda i: (0, 0)),
            pl.BlockSpec((T, 128), lambda i: (0, 0)),
            pl.BlockSpec((1, 128), lambda i: (0, 0)),
        ],
        out_shape=[
            jax.ShapeDtypeStruct((T, 128), jnp.float32),
            jax.ShapeDtypeStruct((T, 128), jnp.int32),
            jax.ShapeDtypeStruct((T, 128), jnp.int32),
            jax.ShapeDtypeStruct((T, 128), jnp.float32),
            jax.ShapeDtypeStruct((T, 128), jnp.float32),
            jax.ShapeDtypeStruct((1, 128), jnp.float32),
        ],
    )(h2, wr_pad)
    return outs


BMOE = 256
PMAX = 2 * T + E * BMOE
NBLK = PMAX // BMOE


def _moe_ffn_kernel(gid_ref, tok_ref, act_ref, h_ref, w1_ref, b1_ref,
                    w2_ref, b2_ref, o_ref, xg_ref):
    i = pl.program_id(0)
    h = pl.program_id(1)
    active = act_ref[i] == 1

    @pl.when(active & (h == 0))
    def _gather():
        def body(r, _):
            t = tok_ref[i * BMOE + r]
            xg_ref[pl.ds(r, 1), :] = h_ref[pl.ds(t, 1), :]
            return 0
        jax.lax.fori_loop(0, BMOE, body, 0, unroll=8)
        o_ref[...] = jnp.broadcast_to(b2_ref[0], (BMOE, C))

    @pl.when(active)
    def _compute():
        a = jnp.dot(xg_ref[...], w1_ref[0],
                    preferred_element_type=jnp.float32) + b1_ref[0]
        a = _gelu_exact(a)
        o_ref[...] += jnp.dot(a, w2_ref[0], preferred_element_type=jnp.float32)


def _moe_ffn(h2, W1, b1, W2, b2, gid, tok, act):
    grid = (NBLK, HID // BH)
    return pl.pallas_call(
        _moe_ffn_kernel,
        grid_spec=pltpu.PrefetchScalarGridSpec(
            num_scalar_prefetch=3,
            grid=grid,
            in_specs=[
                pl.BlockSpec((T, C), lambda i, h, g, tk, ac: (0, 0)),
                pl.BlockSpec((1, C, BH), lambda i, h, g, tk, ac: (g[i], 0, h)),
                pl.BlockSpec((1, 1, BH), lambda i, h, g, tk, ac: (g[i], 0, h)),
                pl.BlockSpec((1, BH, C), lambda i, h, g, tk, ac: (g[i], h, 0)),
                pl.BlockSpec((1, 1, C), lambda i, h, g, tk, ac: (g[i], 0, 0)),
            ],
            out_specs=pl.BlockSpec((BMOE, C), lambda i, h, g, tk, ac: (i, 0)),
            scratch_shapes=[pltpu.VMEM((BMOE, C), jnp.float32)],
        ),
        out_shape=jax.ShapeDtypeStruct((PMAX, C), jnp.float32),
    )(gid, tok, act, h2, W1, b1.reshape(E, 1, HID), W2, b2.reshape(E, 1, C))


def _combine_kernel(p0_ref, p1_ref, x2_ref, w1_ref, w2_ref, eo_ref, o_ref):
    i = pl.program_id(0)

    def body(t, _):
        p0 = p0_ref[i * BM + t]
        p1 = p1_ref[i * BM + t]
        o_ref[pl.ds(t, 1), :] = (
            x2_ref[pl.ds(t, 1), :]
            + w1_ref[pl.ds(t, 1), 0:1] * eo_ref[pl.ds(p0, 1), :]
            + w2_ref[pl.ds(t, 1), 0:1] * eo_ref[pl.ds(p1, 1), :]
        )
        return 0

    jax.lax.fori_loop(0, BM, body, 0, unroll=8)


def _combine(x2, w1b, w2b, eo, pos0, pos1):
    return pl.pallas_call(
        _combine_kernel,
        grid_spec=pltpu.PrefetchScalarGridSpec(
            num_scalar_prefetch=2,
            grid=(T // BM,),
            in_specs=[
                pl.BlockSpec((BM, C), lambda i, p0, p1: (i, 0)),
                pl.BlockSpec((BM, 128), lambda i, p0, p1: (i, 0)),
                pl.BlockSpec((BM, 128), lambda i, p0, p1: (i, 0)),
                pl.BlockSpec((PMAX, C), lambda i, p0, p1: (0, 0)),
            ],
            out_specs=pl.BlockSpec((BM, C), lambda i, p0, p1: (i, 0)),
        ),
        out_shape=jax.ShapeDtypeStruct((T, C), jnp.float32),
        compiler_params=pltpu.CompilerParams(
            vmem_limit_bytes=112 * 1024 * 1024),
    )(pos0, pos1, x2, w1b, w2b, eo)


def _routing_plan(i1, i2):
    eids = jnp.arange(E, dtype=jnp.int32)
    oh1 = (i1[:, None] == eids[None, :]).astype(jnp.int32)
    oh2 = (i2[:, None] == eids[None, :]).astype(jnp.int32)
    c0 = jnp.sum(oh1, axis=0)
    c1 = jnp.sum(oh2, axis=0)
    rank0 = jnp.sum((jnp.cumsum(oh1, axis=0) - 1) * oh1, axis=1)
    rank1 = jnp.sum((jnp.cumsum(oh2, axis=0) - 1) * oh2, axis=1)
    cnt = c0 + c1
    pc = ((cnt + BMOE - 1) // BMOE) * BMOE
    base = jnp.concatenate([jnp.zeros((1,), jnp.int32),
                            jnp.cumsum(pc)[:-1].astype(jnp.int32)])
    pos0 = base[i1] + rank0
    pos1 = base[i2] + c0[i2] + rank1
    ar = jnp.arange(T, dtype=jnp.int32)
    tok = jnp.zeros((PMAX,), jnp.int32).at[pos0].set(ar).at[pos1].set(ar)
    nact = jnp.sum(pc) // BMOE
    blk = jnp.arange(NBLK, dtype=jnp.int32)
    gid = (jnp.searchsorted(base // BMOE, blk, side="right") - 1).astype(
        jnp.int32)
    gid = jnp.clip(gid, 0, E - 1)
    act = (blk < nact).astype(jnp.int32)
    return gid, tok, act, pos0.astype(jnp.int32), pos1.astype(jnp.int32)


def kernel(x, ln1_g, ln1_b, W_attn, b_attn, W_proj, b_proj, ln2_g, ln2_b,
           W_r, W1, b1, W2, b2):
    xf = x.reshape(T, C)
    qkv = _ln_matmul(xf, ln1_g, ln1_b, W_attn, b_attn, BN_QKV)
    q, k, v = jnp.split(qkv, 3, axis=1)
    q = q.reshape(T, NH, DH).transpose(1, 0, 2)
    k = k.reshape(T, NH, DH).transpose(1, 0, 2)
    v = v.reshape(T, NH, DH).transpose(1, 0, 2)
    y = _attention(q, k, v)
    y = y.transpose(1, 0, 2).reshape(T, C)
    x2 = _proj_residual(y, W_proj, b_proj, xf)

    h2 = _layernorm2(x2, ln2_g, ln2_b)
    w8, i1b, i2b, w1b, w2b, cnt = _router(h2, W_r)
    gid, tok, act, pos0, pos1 = _routing_plan(i1b[:, 0], i2b[:, 0])
    eo = _moe_ffn(h2, W1, b1, W2, b2, gid, tok, act)
    out = _combine(x2, w1b, w2b, eo, pos0, pos1)

    counts = cnt[0, :E]
    util = counts / (jnp.sum(counts) + 1e-9)
    return out.reshape(B, T, C), util


# speedup vs baseline: 1.5628x; 1.0089x over previous
import functools

import jax
import jax.numpy as jnp
from jax.experimental import pallas as pl
from jax.experimental.pallas import tpu as pltpu

B, T, C = 1, 2048, 2048
NH = 16
DH = C // NH
E = 8
HID = 4096
TOPK = 2

BM = 256
BN_QKV = 768
BN_PROJ = 512
BH = 1024

NEG = -1e30


def _gelu_exact(a):
    return 0.5 * a * (1.0 + jax.lax.erf(a * (2.0 ** -0.5)))


def _ln(xb, g, b):
    mu = jnp.mean(xb, axis=-1, keepdims=True)
    var = jnp.mean((xb - mu) ** 2, axis=-1, keepdims=True)
    return (xb - mu) / jnp.sqrt(var + 1e-5) * g + b


def _ln_mm_kernel(x_ref, g_ref, b_ref, w_ref, bias_ref, o_ref):
    h = _ln(x_ref[...], g_ref[...], b_ref[...])
    o_ref[...] = (
        jnp.dot(h, w_ref[...], preferred_element_type=jnp.float32)
        + bias_ref[...]
    )


def _ln_matmul(x, g, b, w, bias, bn):
    n = w.shape[1]
    grid = (T // BM, n // bn)
    return pl.pallas_call(
        _ln_mm_kernel,
        grid=grid,
        in_specs=[
            pl.BlockSpec((BM, C), lambda i, j: (i, 0)),
            pl.BlockSpec((1, C), lambda i, j: (0, 0)),
            pl.BlockSpec((1, C), lambda i, j: (0, 0)),
            pl.BlockSpec((C, bn), lambda i, j: (0, j)),
            pl.BlockSpec((1, bn), lambda i, j: (0, j)),
        ],
        out_specs=pl.BlockSpec((BM, bn), lambda i, j: (i, j)),
        out_shape=jax.ShapeDtypeStruct((T, n), jnp.float32),
    )(x, g.reshape(1, C), b.reshape(1, C), w, bias.reshape(1, n))


def _attn_kernel(q_ref, k_ref, v_ref, o_ref):
    i = pl.program_id(1)
    q = q_ref[0]
    k = k_ref[0]
    v = v_ref[0]
    s = jax.lax.dot_general(
        q, k, (((1,), (1,)), ((), ())), preferred_element_type=jnp.float32
    ) * (1.0 / (DH ** 0.5))
    rows = i * BM + jax.lax.broadcasted_iota(jnp.int32, (BM, T), 0)
    cols = jax.lax.broadcasted_iota(jnp.int32, (BM, T), 1)
    s = jnp.where(cols <= rows, s, NEG)
    m = jnp.max(s, axis=-1, keepdims=True)
    pu = jnp.exp(s - m)
    den = jnp.sum(pu, axis=-1, keepdims=True)
    o_ref[0] = jnp.dot(pu, v, preferred_element_type=jnp.float32) / den


def _attention(q, k, v):
    grid = (NH, T // BM)
    return pl.pallas_call(
        _attn_kernel,
        grid=grid,
        in_specs=[
            pl.BlockSpec((1, BM, DH), lambda h, i: (h, i, 0)),
            pl.BlockSpec((1, T, DH), lambda h, i: (h, 0, 0)),
            pl.BlockSpec((1, T, DH), lambda h, i: (h, 0, 0)),
        ],
        out_specs=pl.BlockSpec((1, BM, DH), lambda h, i: (h, i, 0)),
        out_shape=jax.ShapeDtypeStruct((NH, T, DH), jnp.float32),
    )(q, k, v)


def _mm_res_kernel(y_ref, w_ref, b_ref, x_ref, o_ref):
    o_ref[...] = (
        jnp.dot(y_ref[...], w_ref[...], preferred_element_type=jnp.float32)
        + b_ref[...]
        + x_ref[...]
    )


def _proj_residual(y, w, b, x):
    grid = (T // BM, C // BN_PROJ)
    return pl.pallas_call(
        _mm_res_kernel,
        grid=grid,
        in_specs=[
            pl.BlockSpec((BM, C), lambda i, j: (i, 0)),
            pl.BlockSpec((C, BN_PROJ), lambda i, j: (0, j)),
            pl.BlockSpec((1, BN_PROJ), lambda i, j: (0, j)),
            pl.BlockSpec((BM, BN_PROJ), lambda i, j: (i, j)),
        ],
        out_specs=pl.BlockSpec((BM, BN_PROJ), lambda i, j: (i, j)),
        out_shape=jax.ShapeDtypeStruct((T, C), jnp.float32),
    )(y, w, b.reshape(1, C), x)


def _ln_kernel(x_ref, g_ref, b_ref, o_ref):
    o_ref[...] = _ln(x_ref[...], g_ref[...], b_ref[...])


def _layernorm2(x, g, b):
    return pl.pallas_call(
        _ln_kernel,
        grid=(T // BM,),
        in_specs=[
            pl.BlockSpec((BM, C), lambda i: (i, 0)),
            pl.BlockSpec((1, C), lambda i: (0, 0)),
            pl.BlockSpec((1, C), lambda i: (0, 0)),
        ],
        out_specs=pl.BlockSpec((BM, C), lambda i: (i, 0)),
        out_shape=jax.ShapeDtypeStruct((T, C), jnp.float32),
    )(x, g.reshape(1, C), b.reshape(1, C))


def _router_kernel(h_ref, wr_ref, w8_ref, i1_ref, i2_ref, w1_ref, w2_ref,
                   cnt_ref):
    logits = jnp.dot(h_ref[...], wr_ref[...],
                     preferred_element_type=jnp.float32)
    lane = jax.lax.broadcasted_iota(jnp.int32, (T, 128), 1)
    valid = lane < E
    logits = jnp.where(valid, logits, NEG)
    mx = jnp.max(logits, axis=-1, keepdims=True)
    ex = jnp.exp(logits - mx)
    prob = ex / jnp.sum(ex, axis=-1, keepdims=True)

    big = jnp.int32(10**9)
    m1 = jnp.max(prob, axis=-1, keepdims=True)
    i1 = jnp.min(jnp.where(valid & (prob == m1), lane, big), axis=-1,
                 keepdims=True)
    p2 = jnp.where(lane == i1, -1.0, prob)
    m2 = jnp.max(p2, axis=-1, keepdims=True)
    i2 = jnp.min(jnp.where(valid & (p2 == m2), lane, big), axis=-1,
                 keepdims=True)

    denom = m1 + m2 + 1e-9
    w1 = m1 / denom
    w2 = m2 / denom
    sel1 = lane == i1
    sel2 = lane == i2
    w8_ref[...] = jnp.where(sel1, w1, 0.0) + jnp.where(sel2, w2, 0.0)
    i1_ref[...] = jnp.broadcast_to(i1, (T, 128))
    i2_ref[...] = jnp.broadcast_to(i2, (T, 128))
    w1_ref[...] = jnp.broadcast_to(w1, (T, 128))
    w2_ref[...] = jnp.broadcast_to(w2, (T, 128))
    onehot = sel1.astype(jnp.float32) + sel2.astype(jnp.float32)
    cnt_ref[...] = jnp.sum(onehot, axis=0, keepdims=True)


def _router(h2, w_r):
    wr_pad = jnp.zeros((C, 128), jnp.float32).at[:, :E].set(w_r)
    outs = pl.pallas_call(
        _router_kernel,
        grid=(1,),
        in_specs=[
            pl.BlockSpec((T, C), lambda i: (0, 0)),
            pl.BlockSpec((C, 128), lambda i: (0, 0)),
        ],
        out_specs=[
            pl.BlockSpec((T, 128), lambda i: (0, 0)),
            pl.BlockSpec((T, 128), lambda i: (0, 0)),
            pl.BlockSpec((T, 128), lambda i: (0, 0)),
            pl.BlockSpec((T, 128), lambda i: (0, 0)),
            pl.BlockSpec((T, 128), lambda i: (0, 0)),
            pl.BlockSpec((1, 128), lambda i: (0, 0)),
        ],
        out_shape=[
            jax.ShapeDtypeStruct((T, 128), jnp.float32),
            jax.ShapeDtypeStruct((T, 128), jnp.int32),
            jax.ShapeDtypeStruct((T, 128), jnp.int32),
            jax.ShapeDtypeStruct((T, 128), jnp.float32),
            jax.ShapeDtypeStruct((T, 128), jnp.float32),
            jax.ShapeDtypeStruct((1, 128), jnp.float32),
        ],
    )(h2, wr_pad)
    return outs


BMOE = 256
PMAX = 2 * T + E * BMOE
NBLK = PMAX // BMOE


def _moe_ffn_kernel(gid_ref, tok_ref, act_ref, h_ref, w1_ref, b1_ref,
                    w2_ref, b2_ref, o_ref, xg_ref):
    i = pl.program_id(0)
    h = pl.program_id(1)
    active = act_ref[i] == 1

    @pl.when(active & (h == 0))
    def _gather():
        def body(r, _):
            t = tok_ref[i * BMOE + r]
            xg_ref[pl.ds(r, 1), :] = h_ref[pl.ds(t, 1), :]
            return 0
        jax.lax.fori_loop(0, BMOE, body, 0, unroll=8)
        o_ref[...] = jnp.broadcast_to(b2_ref[0], (BMOE, C))

    @pl.when(active)
    def _compute():
        a = jnp.dot(xg_ref[...], w1_ref[0],
                    preferred_element_type=jnp.float32) + b1_ref[0]
        a = _gelu_exact(a)
        o_ref[...] += jnp.dot(a, w2_ref[0], preferred_element_type=jnp.float32)


def _moe_ffn(h2, W1, b1, W2, b2, gid, tok, act):
    grid = (NBLK, HID // BH)
    return pl.pallas_call(
        _moe_ffn_kernel,
        grid_spec=pltpu.PrefetchScalarGridSpec(
            num_scalar_prefetch=3,
            grid=grid,
            in_specs=[
                pl.BlockSpec((T, C), lambda i, h, g, tk, ac: (0, 0)),
                pl.BlockSpec((1, C, BH), lambda i, h, g, tk, ac: (g[i], 0, h)),
                pl.BlockSpec((1, 1, BH), lambda i, h, g, tk, ac: (g[i], 0, h)),
                pl.BlockSpec((1, BH, C), lambda i, h, g, tk, ac: (g[i], h, 0)),
                pl.BlockSpec((1, 1, C), lambda i, h, g, tk, ac: (g[i], 0, 0)),
            ],
            out_specs=pl.BlockSpec((BMOE, C), lambda i, h, g, tk, ac: (i, 0)),
            scratch_shapes=[pltpu.VMEM((BMOE, C), jnp.float32)],
        ),
        out_shape=jax.ShapeDtypeStruct((PMAX, C), jnp.float32),
    )(gid, tok, act, h2, W1, b1.reshape(E, 1, HID), W2, b2.reshape(E, 1, C))


def _combine_kernel(p0_ref, p1_ref, x2_ref, w1_ref, w2_ref, eo_ref, o_ref):
    i = pl.program_id(0)

    def body(t, _):
        p0 = p0_ref[i * BM + t]
        p1 = p1_ref[i * BM + t]
        o_ref[pl.ds(t, 1), :] = (
            x2_ref[pl.ds(t, 1), :]
            + w1_ref[pl.ds(t, 1), 0:1] * eo_ref[pl.ds(p0, 1), :]
            + w2_ref[pl.ds(t, 1), 0:1] * eo_ref[pl.ds(p1, 1), :]
        )
        return 0

    jax.lax.fori_loop(0, BM, body, 0, unroll=8)


def _combine(x2, w1b, w2b, eo, pos0, pos1):
    return pl.pallas_call(
        _combine_kernel,
        grid_spec=pltpu.PrefetchScalarGridSpec(
            num_scalar_prefetch=2,
            grid=(T // BM,),
            in_specs=[
                pl.BlockSpec((BM, C), lambda i, p0, p1: (i, 0)),
                pl.BlockSpec((BM, 128), lambda i, p0, p1: (i, 0)),
                pl.BlockSpec((BM, 128), lambda i, p0, p1: (i, 0)),
                pl.BlockSpec((PMAX, C), lambda i, p0, p1: (0, 0)),
            ],
            out_specs=pl.BlockSpec((BM, C), lambda i, p0, p1: (i, 0)),
        ),
        out_shape=jax.ShapeDtypeStruct((T, C), jnp.float32),
        compiler_params=pltpu.CompilerParams(
            vmem_limit_bytes=112 * 1024 * 1024),
    )(pos0, pos1, x2, w1b, w2b, eo)


def _routing_plan(i1, i2):
    eids = jnp.arange(E, dtype=jnp.int32)
    oh1 = (i1[:, None] == eids[None, :]).astype(jnp.int32)
    oh2 = (i2[:, None] == eids[None, :]).astype(jnp.int32)
    c0 = jnp.sum(oh1, axis=0)
    c1 = jnp.sum(oh2, axis=0)
    rank0 = jnp.sum((jnp.cumsum(oh1, axis=0) - 1) * oh1, axis=1)
    rank1 = jnp.sum((jnp.cumsum(oh2, axis=0) - 1) * oh2, axis=1)
    cnt = c0 + c1
    pc = ((cnt + BMOE - 1) // BMOE) * BMOE
    base = jnp.concatenate([jnp.zeros((1,), jnp.int32),
                            jnp.cumsum(pc)[:-1].astype(jnp.int32)])
    pos0 = base[i1] + rank0
    pos1 = base[i2] + c0[i2] + rank1
    ar = jnp.arange(T, dtype=jnp.int32)
    tok = jnp.zeros((PMAX,), jnp.int32).at[pos0].set(ar).at[pos1].set(ar)
    nact = jnp.sum(pc) // BMOE
    blk = jnp.arange(NBLK, dtype=jnp.int32)
    gid = (jnp.searchsorted(base // BMOE, blk, side="right") - 1).astype(
        jnp.int32)
    gid = jnp.clip(gid, 0, E - 1)
    act = (blk < nact).astype(jnp.int32)
    return gid, tok, act, pos0.astype(jnp.int32), pos1.astype(jnp.int32)


def kernel(x, ln1_g, ln1_b, W_attn, b_attn, W_proj, b_proj, ln2_g, ln2_b,
           W_r, W1, b1, W2, b2):
    xf = x.reshape(T, C)
    qkv = _ln_matmul(xf, ln1_g, ln1_b, W_attn, b_attn, BN_QKV)
    q, k, v = jnp.split(qkv, 3, axis=1)
    q = q.reshape(T, NH, DH).transpose(1, 0, 2)
    k = k.reshape(T, NH, DH).transpose(1, 0, 2)
    v = v.reshape(T, NH, DH).transpose(1, 0, 2)
    y = _attention(q, k, v)
    y = y.transpose(1, 0, 2).reshape(T, C)
    x2 = _proj_residual(y, W_proj, b_proj, xf)

    h2 = _layernorm2(x2, ln2_g, ln2_b)
    w8, i1b, i2b, w1b, w2b, cnt = _router(h2, W_r)
    gid, tok, act, pos0, pos1 = _routing_plan(i1b[:, 0], i2b[:, 0])
    eo = _moe_ffn(h2, W1, b1, W2, b2, gid, tok, act)
    out = _combine(x2, w1b, w2b, eo, pos0, pos1)

    counts = cnt[0, :E]
    util = counts / (jnp.sum(counts) + 1e-9)
    return out.reshape(B, T, C), util
